# baseline (speedup 1.0000x reference)
import functools

import jax
import jax.numpy as jnp
from jax import lax
from jax.experimental import pallas as pl
from jax.experimental.pallas import tpu as pltpu
from jax.experimental.pallas import tpu_sc as plsc

LAMBDA = 0.0001
B = 16384
D = 32
NC, NS, L = 2, 16, 16
NW = NC * NS
BPW = B // NW
CHUNK = 128
NCH = BPW // CHUNK
K = 16
KC = CHUNK // K
GPC = CHUNK // L


def _sc_user_body(user_hbm, eu_hbm, urows_hbm, ui, buf, sem):
    wid = lax.axis_index("s") * NC + lax.axis_index("c")
    base = wid * BPW
    pltpu.sync_copy(user_hbm.at[pl.ds(base, BPW)], ui)
    cps = []
    for c in range(NCH):
        sl = pl.ds(c * CHUNK, CHUNK)
        cps.append(pltpu.async_copy(eu_hbm.at[ui.at[sl]], buf.at[sl], sem))
    for cp in cps:
        cp.wait()
    pltpu.sync_copy(buf, urows_hbm.at[pl.ds(base, BPW), :])


_sc_gather_user = functools.partial(
    pl.kernel,
    out_type=jax.ShapeDtypeStruct((B, D), jnp.float32),
    mesh=plsc.VectorSubcoreMesh(core_axis_name="c", subcore_axis_name="s"),
    compiler_params=pltpu.CompilerParams(
        needs_layout_passes=False, use_tc_tiling_on_sc=False),
    scratch_types=[
        pltpu.VMEM((BPW,), jnp.int32),
        pltpu.VMEM((BPW, D), jnp.float32),
        pltpu.SemaphoreType.DMA,
    ],
)(_sc_user_body)


def _sc_items_body(itemi_hbm, itemj_hbm, urows_hbm, ei_hbm,
                   x_hbm, sums_hbm,
                   ii, ji, ubuf, ibuf, jbuf, xbuf, sbuf, s0, s1):
    wid = lax.axis_index("s") * NC + lax.axis_index("c")
    base = wid * BPW

    pltpu.sync_copy(itemi_hbm.at[pl.ds(base, BPW)], ii.at[pl.ds(0, BPW)])
    pltpu.sync_copy(itemj_hbm.at[pl.ds(base, BPW)], ji.at[pl.ds(0, BPW)])

    def fire_k(gb, lb, sem):
        def row(k, _):
            ri = ii[pl.ds(gb + k, L)][0]
            rj = ji[pl.ds(gb + k, L)][0]
            pltpu.async_copy(ei_hbm.at[pl.ds(ri, 1), :],
                             ibuf.at[pl.ds(lb + k, 1), :], sem)
            pltpu.async_copy(ei_hbm.at[pl.ds(rj, 1), :],
                             jbuf.at[pl.ds(lb + k, 1), :], sem)
            return 0
        lax.fori_loop(0, K, row, 0)

    def drain_k(lb, sem):
        pltpu.make_async_copy(ei_hbm.at[pl.ds(0, K), :],
                              ibuf.at[pl.ds(lb, K), :], sem).wait()
        pltpu.make_async_copy(ei_hbm.at[pl.ds(0, K), :],
                              jbuf.at[pl.ds(lb, K), :], sem).wait()

    lanes = lax.iota(jnp.int32, L)
    zeros = jnp.zeros((L,), jnp.float32)
    su = si = sj = zeros

    for c in range(NCH):
        ucp = pltpu.async_copy(
            urows_hbm.at[pl.ds(base + c * CHUNK, CHUNK), :], ubuf, s1)
        fire_k(c * CHUNK, 0, s0)
        fire_k(c * CHUNK + K, K, s0)

        def wave(q, _):
            fire_k(c * CHUNK + (q + 2) * K, (q + 2) * K, s0)
            drain_k(q * K, s0)
            return 0
        lax.fori_loop(0, KC - 2, wave, 0)
        drain_k((KC - 2) * K, s0)
        drain_k((KC - 1) * K, s0)
        ucp.wait()

        def group(g, carry):
            su, si, sj = carry
            rvec = g * L + lanes
            acc_i = zeros
            acc_j = zeros
            for d in range(D):
                dvec = jnp.full((L,), d, jnp.int32)
                uu = plsc.load_gather(ubuf, [rvec, dvec])
                vi = plsc.load_gather(ibuf, [rvec, dvec])
                vj = plsc.load_gather(jbuf, [rvec, dvec])
                acc_i = acc_i + uu * vi
                acc_j = acc_j + uu * vj
                su = su + uu * uu
                si = si + vi * vi
                sj = sj + vj * vj
            xbuf[pl.ds(c * CHUNK + g * L, L)] = acc_i - acc_j
            return su, si, sj

        su, si, sj = lax.fori_loop(0, GPC, group, (su, si, sj))

    sbuf[pl.ds(0, L)] = su
    sbuf[pl.ds(L, L)] = si
    sbuf[pl.ds(2 * L, L)] = sj
    pltpu.sync_copy(xbuf, x_hbm.at[pl.ds(base, BPW)])
    pltpu.sync_copy(sbuf, sums_hbm.at[pl.ds(wid * 3 * L, 3 * L)])


_sc_items_dots = functools.partial(
    pl.kernel,
    out_type=[jax.ShapeDtypeStruct((B,), jnp.float32),
              jax.ShapeDtypeStruct((NW * 3 * L,), jnp.float32)],
    mesh=plsc.VectorSubcoreMesh(core_axis_name="c", subcore_axis_name="s"),
    compiler_params=pltpu.CompilerParams(
        needs_layout_passes=False, use_tc_tiling_on_sc=True),
    scratch_types=[
        pltpu.VMEM((BPW + L,), jnp.int32),
        pltpu.VMEM((BPW + L,), jnp.int32),
        pltpu.VMEM((CHUNK, D), jnp.float32),
        pltpu.VMEM((CHUNK, D), jnp.float32),
        pltpu.VMEM((CHUNK, D), jnp.float32),
        pltpu.VMEM((BPW,), jnp.float32),
        pltpu.VMEM((3 * L,), jnp.float32),
        pltpu.SemaphoreType.DMA,
        pltpu.SemaphoreType.DMA,
    ],
)(_sc_items_body)


def _tc_body(x_ref, s_ref, o_ref):
    x = x_ref[...]
    sp = jnp.maximum(-x, 0.0) + jnp.log1p(jnp.exp(-jnp.abs(x)))
    l2 = LAMBDA * jnp.sum(s_ref[...]) / (B * D)
    o_ref[0, 0] = jnp.sum(sp) / B + l2


_tc_loss = pl.pallas_call(
    _tc_body,
    out_shape=jax.ShapeDtypeStruct((1, 1), jnp.float32),
    in_specs=[pl.BlockSpec((128, 128), lambda: (0, 0)),
              pl.BlockSpec((NW * 3 * L,), lambda: (0,))],
    out_specs=pl.BlockSpec(memory_space=pltpu.SMEM),
)


def kernel(user, item_i, item_j, embed_user, embed_item):
    u_rows = _sc_gather_user(user, embed_user)
    x, sums = _sc_items_dots(item_i, item_j, u_rows, embed_item)
    out = _tc_loss(x.reshape(128, 128), sums)
    return out[0, 0]


# speedup vs baseline: 1.3405x; 1.3405x over previous
import functools

import jax
import jax.numpy as jnp
from jax import lax
from jax.experimental import pallas as pl
from jax.experimental.pallas import tpu as pltpu
from jax.experimental.pallas import tpu_sc as plsc

LAMBDA = 0.0001
B = 16384
D = 32
NC, NS, L = 2, 16, 16
NW = NC * NS
BPW = B // NW
CHUNK = 128
NCH = BPW // CHUNK
K = 16
KC = CHUNK // K
GPC = CHUNK // L


def _sc_body(user_hbm, itemi_hbm, itemj_hbm, eu_hbm, ei_hbm,
             x_hbm, sums_hbm,
             ui, ii, ji, ubuf, ibuf, jbuf, xbuf, sbuf, s0, s1):
    wid = lax.axis_index("s") * NC + lax.axis_index("c")
    base = wid * BPW

    pltpu.sync_copy(user_hbm.at[pl.ds(base, BPW)], ui.at[pl.ds(0, BPW)])
    pltpu.sync_copy(itemi_hbm.at[pl.ds(base, BPW)], ii.at[pl.ds(0, BPW)])
    pltpu.sync_copy(itemj_hbm.at[pl.ds(base, BPW)], ji.at[pl.ds(0, BPW)])

    def fire_k(gb, lb, sem):
        def row(k, _):
            ru = ui[pl.ds(gb + k, L)][0]
            ri = ii[pl.ds(gb + k, L)][0]
            rj = ji[pl.ds(gb + k, L)][0]
            pltpu.async_copy(eu_hbm.at[pl.ds(ru, 1), :],
                             ubuf.at[pl.ds(lb + k, 1), :], sem)
            pltpu.async_copy(ei_hbm.at[pl.ds(ri, 1), :],
                             ibuf.at[pl.ds(lb + k, 1), :], sem)
            pltpu.async_copy(ei_hbm.at[pl.ds(rj, 1), :],
                             jbuf.at[pl.ds(lb + k, 1), :], sem)
            return 0
        lax.fori_loop(0, K, row, 0)

    def drain_k(lb, sem):
        pltpu.make_async_copy(eu_hbm.at[pl.ds(0, K), :],
                              ubuf.at[pl.ds(lb, K), :], sem).wait()
        pltpu.make_async_copy(ei_hbm.at[pl.ds(0, K), :],
                              ibuf.at[pl.ds(lb, K), :], sem).wait()
        pltpu.make_async_copy(ei_hbm.at[pl.ds(0, K), :],
                              jbuf.at[pl.ds(lb, K), :], sem).wait()

    lanes = lax.iota(jnp.int32, L)
    zeros = jnp.zeros((L,), jnp.float32)
    su = si = sj = zeros

    for c in range(NCH):
        fire_k(c * CHUNK, 0, s0)
        fire_k(c * CHUNK + K, K, s0)

        def wave(q, _):
            fire_k(c * CHUNK + (q + 2) * K, (q + 2) * K, s0)
            drain_k(q * K, s0)
            return 0
        lax.fori_loop(0, KC - 2, wave, 0)
        drain_k((KC - 2) * K, s0)
        drain_k((KC - 1) * K, s0)

        def group(g, carry):
            su, si, sj = carry
            rvec = g * L + lanes
            acc_i = zeros
            acc_j = zeros
            for d in range(D):
                dvec = jnp.full((L,), d, jnp.int32)
                uu = plsc.load_gather(ubuf, [rvec, dvec])
                vi = plsc.load_gather(ibuf, [rvec, dvec])
                vj = plsc.load_gather(jbuf, [rvec, dvec])
                acc_i = acc_i + uu * vi
                acc_j = acc_j + uu * vj
                su = su + uu * uu
                si = si + vi * vi
                sj = sj + vj * vj
            xbuf[pl.ds(c * CHUNK + g * L, L)] = acc_i - acc_j
            return su, si, sj

        su, si, sj = lax.fori_loop(0, GPC, group, (su, si, sj))

    sbuf[pl.ds(0, L)] = su
    sbuf[pl.ds(L, L)] = si
    sbuf[pl.ds(2 * L, L)] = sj
    pltpu.sync_copy(xbuf, x_hbm.at[pl.ds(base, BPW)])
    pltpu.sync_copy(sbuf, sums_hbm.at[pl.ds(wid * 3 * L, 3 * L)])


_sc_gather_dots = functools.partial(
    pl.kernel,
    out_type=[jax.ShapeDtypeStruct((B,), jnp.float32),
              jax.ShapeDtypeStruct((NW * 3 * L,), jnp.float32)],
    mesh=plsc.VectorSubcoreMesh(core_axis_name="c", subcore_axis_name="s"),
    compiler_params=pltpu.CompilerParams(
        needs_layout_passes=False, use_tc_tiling_on_sc=True),
    scratch_types=[
        pltpu.VMEM((BPW + L,), jnp.int32),
        pltpu.VMEM((BPW + L,), jnp.int32),
        pltpu.VMEM((BPW + L,), jnp.int32),
        pltpu.VMEM((CHUNK, D), jnp.float32),
        pltpu.VMEM((CHUNK, D), jnp.float32),
        pltpu.VMEM((CHUNK, D), jnp.float32),
        pltpu.VMEM((BPW,), jnp.float32),
        pltpu.VMEM((3 * L,), jnp.float32),
        pltpu.SemaphoreType.DMA,
        pltpu.SemaphoreType.DMA,
    ],
)(_sc_body)


def _tc_body(x_ref, s_ref, o_ref):
    x = x_ref[...]
    sp = jnp.maximum(-x, 0.0) + jnp.log1p(jnp.exp(-jnp.abs(x)))
    l2 = LAMBDA * jnp.sum(s_ref[...]) / (B * D)
    o_ref[0, 0] = jnp.sum(sp) / B + l2


_tc_loss = pl.pallas_call(
    _tc_body,
    out_shape=jax.ShapeDtypeStruct((1, 1), jnp.float32),
    in_specs=[pl.BlockSpec((128, 128), lambda: (0, 0)),
              pl.BlockSpec((NW * 3 * L,), lambda: (0,))],
    out_specs=pl.BlockSpec(memory_space=pltpu.SMEM),
)


def kernel(user, item_i, item_j, embed_user, embed_item):
    x, sums = _sc_gather_dots(user, item_i, item_j, embed_user, embed_item)
    out = _tc_loss(x.reshape(128, 128), sums)
    return out[0, 0]
